# baseline (speedup 1.0000x reference)
import functools

import jax
import jax.numpy as jnp
from jax import lax
from jax.experimental import pallas as pl
from jax.experimental.pallas import tpu as pltpu
from jax.experimental.pallas import tpu_sc as plsc

K = 8
ROWS = 512
TBL = 32


def _silu(x):
    return x * jax.nn.sigmoid(x)


def _embed_body(len_ref, res_ref, coords_ref, tok_ref, pos_ref,
                out_ref, ct_ref):
    b = pl.program_id(0)
    iblk = pl.program_id(1)
    len_b = len_ref[b]
    res = res_ref[0]
    ntok = tok_ref.shape[0]
    f = jnp.where(res == ntok - 2, tok_ref[ntok - 2:ntok - 1, :],
                  tok_ref[ntok - 1:ntok, :])
    for t in range(ntok - 3, -1, -1):
        f = jnp.where(res == t, tok_ref[t:t + 1, :], f)
    feats = f + pos_ref[...]
    R = feats.shape[0]
    ii = iblk * R + lax.broadcasted_iota(jnp.int32, (R, 1), 0)
    maskc = (ii < len_b).astype(jnp.float32)
    pad = jnp.zeros((R, TBL - 20), jnp.float32)
    coors = coords_ref[0]
    out_ref[0] = jnp.concatenate([feats, coors, maskc, pad], axis=1)
    ct_ref[0] = jnp.swapaxes(coors, 0, 1)


def _embed(lengths, residues, coords, tok, pos):
    B, L = residues.shape
    emb = tok.shape[1]
    res3 = residues.reshape(B, L, 1).astype(jnp.int32)
    grid = (B, L // ROWS)
    return pl.pallas_call(
        _embed_body,
        grid=grid,
        in_specs=[
            pl.BlockSpec(memory_space=pltpu.SMEM),
            pl.BlockSpec((1, ROWS, 1), lambda b, i: (b, i, 0)),
            pl.BlockSpec((1, ROWS, 3), lambda b, i: (b, i, 0)),
            pl.BlockSpec(tok.shape, lambda b, i: (0, 0)),
            pl.BlockSpec((ROWS, emb), lambda b, i: (i, 0)),
        ],
        out_specs=[
            pl.BlockSpec((1, ROWS, TBL), lambda b, i: (b, i, 0)),
            pl.BlockSpec((1, 3, ROWS), lambda b, i: (b, 0, i)),
        ],
        out_shape=[
            jax.ShapeDtypeStruct((B, L, TBL), jnp.float32),
            jax.ShapeDtypeStruct((B, 3, L), jnp.float32),
        ],
    )(lengths, res3, coords, tok, pos)


def _topk_body(len_ref, table_ref, ct_ref, idx_ref):
    b = pl.program_id(0)
    iblk = pl.program_id(1)
    L = ct_ref.shape[2]
    len_b = len_ref[b]

    @pl.when(iblk * ROWS >= len_b)
    def _skip():
        idx_ref[0] = jnp.full((K - 1, ROWS), b * L, jnp.int32)

    @pl.when(iblk * ROWS < len_b)
    def _full():
        t = table_ref[0]
        xr = t[:, 16:19]
        ct = ct_ref[0]
        ii = iblk * ROWS + lax.broadcasted_iota(jnp.int32, (ROWS, 1), 0)
        ji = lax.broadcasted_iota(jnp.int32, (1, L), 1)
        nx = (xr[:, 0:1] * xr[:, 0:1] + xr[:, 1:2] * xr[:, 1:2]
              + xr[:, 2:3] * xr[:, 2:3])
        nj = (ct[0:1, :] * ct[0:1, :] + ct[1:2, :] * ct[1:2, :]
              + ct[2:3, :] * ct[2:3, :])
        dist = (nx + nj) - 2.0 * jnp.dot(xr, ct,
                                         preferred_element_type=jnp.float32)
        valid = (ii < len_b) & (ji < len_b)
        rank = jnp.where(valid, dist, jnp.float32(1e5))
        rank = jnp.where(ii == ji, jnp.float32(jnp.inf), rank)
        adj = (jnp.abs(ii - ji) == 1) & (jnp.maximum(ii, ji) < len_b)
        rank = jnp.where(adj, jnp.float32(0.0), rank)
        jif = ji.astype(jnp.float32)
        cols = []
        for _ in range(K - 1):
            mval = jnp.min(rank, axis=1, keepdims=True)
            amin = jnp.min(jnp.where(rank == mval, jif, jnp.float32(L)),
                           axis=1, keepdims=True)
            cols.append(amin)
            rank = jnp.where(jif == amin, jnp.float32(jnp.inf), rank)
        A = jnp.concatenate(cols, axis=1) + jnp.float32(b * L)
        idx_ref[0] = jnp.swapaxes(A, 0, 1).astype(jnp.int32)


def _topk(lengths, table, ct):
    B, L, _ = table.shape
    grid = (B, L // ROWS)
    return pl.pallas_call(
        _topk_body,
        grid=grid,
        in_specs=[
            pl.BlockSpec(memory_space=pltpu.SMEM),
            pl.BlockSpec((1, ROWS, TBL), lambda b, i: (b, i, 0)),
            pl.BlockSpec((1, 3, L), lambda b, i: (b, 0, 0)),
        ],
        out_specs=pl.BlockSpec((1, K - 1, ROWS), lambda b, i: (b, 0, i)),
        out_shape=jax.ShapeDtypeStruct((B, K - 1, L), jnp.int32),
    )(lengths, table, ct)


def _sc_gather(table_flat, gidx):
    M = gidx.shape[0]
    N, D = table_flat.shape
    info = plsc.get_sparse_core_info()
    nw = info.num_cores * info.num_subcores
    rows_w = M // nw
    ch = 128
    nch = rows_w // ch
    stage = N // info.num_subcores
    mesh = plsc.VectorSubcoreMesh(core_axis_name="c", subcore_axis_name="s")

    @functools.partial(
        pl.kernel,
        out_type=jax.ShapeDtypeStruct((M, D), jnp.float32),
        mesh=mesh,
        compiler_params=pltpu.CompilerParams(use_tc_tiling_on_sc=False),
        scratch_types=[
            pltpu.VMEM((rows_w,), jnp.int32),
            pltpu.VMEM((rows_w, D), jnp.float32),
            pltpu.VMEM_SHARED((N, D), jnp.float32),
            pltpu.SemaphoreType.DMA,
        ],
    )
    def gk(table_hbm, idx_hbm, out_hbm, idx_v, rows_v, shared, sem):
        sid = lax.axis_index("s")
        wid = sid * info.num_cores + lax.axis_index("c")
        base = wid * rows_w
        pltpu.sync_copy(table_hbm.at[pl.ds(sid * stage, stage)],
                        shared.at[pl.ds(sid * stage, stage)])
        pltpu.sync_copy(idx_hbm.at[pl.ds(base, rows_w)], idx_v)
        plsc.subcore_barrier()
        cps = [
            pltpu.async_copy(
                shared.at[idx_v.at[pl.ds(c * ch, ch)]],
                rows_v.at[pl.ds(c * ch, ch)],
                sem,
            )
            for c in range(nch)
        ]
        for cp in cps:
            cp.wait()
        pltpu.sync_copy(rows_v, out_hbm.at[pl.ds(base, rows_w)])

    return gk(table_flat, gidx)


def _mp_body(len_ref, table_ref, g_ref,
             ew1_ref, eb1_ref, ew2_ref, eb2_ref,
             nw1_ref, nb1_ref, nw2_ref, nb2_ref,
             cw1_ref, cb1_ref, cw2_ref, cb2_ref,
             lnw_ref, lnb_ref, *rest, last):
    if last:
        fw_ref, fb_ref, out_ref = rest
    else:
        fw_ref = fb_ref = None
        out_ref, ct_ref = rest
    b = pl.program_id(0)
    iblk = pl.program_id(1)
    len_b = len_ref[b]
    t = table_ref[0]
    feats = t[:, 0:16]
    coors = t[:, 16:19]
    R = feats.shape[0]

    def node_update(m_i, coors_new):
        mu = jnp.mean(feats, axis=1, keepdims=True)
        var = jnp.mean((feats - mu) ** 2, axis=1, keepdims=True)
        normed = ((feats - mu) / jnp.sqrt(var + 1e-5) * lnw_ref[...]
                  + lnb_ref[...])
        node_in = jnp.concatenate([normed, m_i], axis=1)
        h2 = _silu(jnp.dot(node_in, nw1_ref[...],
                           preferred_element_type=jnp.float32) + nb1_ref[...])
        node_out = (jnp.dot(h2, nw2_ref[...],
                            preferred_element_type=jnp.float32)
                    + nb2_ref[...] + feats)
        if last:
            out_ref[0] = (jnp.dot(node_out, fw_ref[...],
                                  preferred_element_type=jnp.float32)
                          + fb_ref[...])
        else:
            ii = iblk * R + lax.broadcasted_iota(jnp.int32, (R, 1), 0)
            maskc = (ii < len_b).astype(jnp.float32)
            pad = jnp.zeros((R, TBL - 20), jnp.float32)
            out_ref[0] = jnp.concatenate([node_out, coors_new, maskc, pad],
                                         axis=1)
            ct_ref[0] = jnp.swapaxes(coors_new, 0, 1)

    @pl.when(iblk * R >= len_b)
    def _skip():
        node_update(jnp.zeros_like(feats), coors)

    @pl.when(iblk * R < len_b)
    def _full():
        ii = iblk * R + lax.broadcasted_iota(jnp.int32, (R, 1), 0)
        mask_i = ii < len_b

        zero1 = jnp.zeros((R, 1), jnp.float32)
        e_parts = [jnp.concatenate([feats, feats, zero1], axis=1)]
        m2_parts = [mask_i]
        rel_parts = []
        for k in range(1, K):
            g = g_ref[0, k - 1]
            fj = g[:, 0:16]
            cj = g[:, 16:19]
            rel = coors - cj
            a = rel[:, 0:1]
            bb = rel[:, 1:2]
            c = rel[:, 2:3]
            rd = a * a + bb * bb + c * c
            e_parts.append(jnp.concatenate([feats, fj, rd], axis=1))
            m2_parts.append(mask_i & (g[:, 19:20] > 0.5))
            rel_parts.append(rel)
        E = jnp.concatenate(e_parts, axis=0)
        M2 = jnp.concatenate(m2_parts, axis=0)

        h = _silu(jnp.dot(E, ew1_ref[...], preferred_element_type=jnp.float32)
                  + eb1_ref[...])
        m = _silu(jnp.dot(h, ew2_ref[...], preferred_element_type=jnp.float32)
                  + eb2_ref[...])
        m = jnp.where(M2, m, jnp.float32(0.0))
        m_i = m[0:R]
        for k in range(1, K):
            m_i = m_i + m[k * R:(k + 1) * R]

        if last:
            coors_new = None
        else:
            REL = jnp.concatenate(rel_parts, axis=0)
            cw = jnp.dot(_silu(jnp.dot(m[R:], cw1_ref[...],
                                       preferred_element_type=jnp.float32)
                               + cb1_ref[...]),
                         cw2_ref[...],
                         preferred_element_type=jnp.float32) + cb2_ref[...]
            cw = jnp.where(M2[R:], cw, jnp.float32(0.0))
            contrib = cw * REL
            coors_new = coors
            for k in range(1, K):
                coors_new = coors_new + contrib[(k - 1) * R:k * R]
        node_update(m_i, coors_new)


def _mp(lengths, table, G, w, final):
    B, L, _ = table.shape
    grid = (B, L // ROWS)
    full = lambda a: pl.BlockSpec(a.shape, lambda b, i: (0,) * a.ndim)
    last = final is not None
    in_specs = [
        pl.BlockSpec(memory_space=pltpu.SMEM),
        pl.BlockSpec((1, ROWS, TBL), lambda b, i: (b, i, 0)),
        pl.BlockSpec((1, K - 1, ROWS, TBL), lambda b, i: (b, 0, i, 0)),
    ] + [full(a) for a in w]
    args = [lengths, table, G, *w]
    if last:
        in_specs += [full(a) for a in final]
        args += list(final)
        out_specs = pl.BlockSpec((1, ROWS, 3), lambda b, i: (b, i, 0))
        out_shape = jax.ShapeDtypeStruct((B, L, 3), jnp.float32)
    else:
        out_specs = [
            pl.BlockSpec((1, ROWS, TBL), lambda b, i: (b, i, 0)),
            pl.BlockSpec((1, 3, ROWS), lambda b, i: (b, 0, i)),
        ]
        out_shape = [
            jax.ShapeDtypeStruct((B, L, TBL), jnp.float32),
            jax.ShapeDtypeStruct((B, 3, L), jnp.float32),
        ]
    return pl.pallas_call(
        functools.partial(_mp_body, last=last),
        grid=grid,
        in_specs=in_specs,
        out_specs=out_specs,
        out_shape=out_shape,
    )(*args)


def kernel(coords, residues, lengths, params):
    B, L, _ = coords.shape
    lengths = lengths.astype(jnp.int32)
    table, ct = _embed(lengths, residues, coords.astype(jnp.float32),
                       params['token_emb'], params['pos_emb'])
    nlayers = len(params['layers'])
    out = None
    for li, lp in enumerate(params['layers']):
        idx_t = _topk(lengths, table, ct)
        G = _sc_gather(table.reshape(B * L, TBL),
                       idx_t.reshape(-1)).reshape(B, K - 1, L, TBL)
        w = (lp['e_w1'], lp['e_b1'].reshape(1, -1),
             lp['e_w2'], lp['e_b2'].reshape(1, -1),
             lp['n_w1'], lp['n_b1'].reshape(1, -1),
             lp['n_w2'], lp['n_b2'].reshape(1, -1),
             lp['c_w1'], lp['c_b1'].reshape(1, -1),
             lp['c_w2'], lp['c_b2'].reshape(1, -1),
             lp['ln_w'].reshape(1, -1), lp['ln_b'].reshape(1, -1))
        if li == nlayers - 1:
            out = _mp(lengths, table, G, w,
                      (params['final_w'], params['final_b'].reshape(1, -1)))
        else:
            table, ct = _mp(lengths, table, G, w, None)
    return out


# speedup vs baseline: 1.1130x; 1.1130x over previous
import functools

import jax
import jax.numpy as jnp
from jax import lax
from jax.experimental import pallas as pl
from jax.experimental.pallas import tpu as pltpu
from jax.experimental.pallas import tpu_sc as plsc

K = 8
ROWS = 512
TBL = 32


def _silu(x):
    return x * jax.nn.sigmoid(x)


def _embed_body(len_ref, res_ref, coords_ref, tok_ref, pos_ref,
                out_ref, ct_ref):
    b = pl.program_id(0)
    iblk = pl.program_id(1)
    len_b = len_ref[b]
    res = res_ref[0]
    ntok = tok_ref.shape[0]
    f = jnp.where(res == ntok - 2, tok_ref[ntok - 2:ntok - 1, :],
                  tok_ref[ntok - 1:ntok, :])
    for t in range(ntok - 3, -1, -1):
        f = jnp.where(res == t, tok_ref[t:t + 1, :], f)
    feats = f + pos_ref[...]
    R = feats.shape[0]
    ii = iblk * R + lax.broadcasted_iota(jnp.int32, (R, 1), 0)
    maskc = (ii < len_b).astype(jnp.float32)
    pad = jnp.zeros((R, TBL - 20), jnp.float32)
    coors = coords_ref[0]
    out_ref[0] = jnp.concatenate([feats, coors, maskc, pad], axis=1)
    ct_ref[0] = jnp.swapaxes(coors, 0, 1)


def _embed(lengths, residues, coords, tok, pos):
    B, L = residues.shape
    emb = tok.shape[1]
    res3 = residues.reshape(B, L, 1).astype(jnp.int32)
    grid = (B, L // ROWS)
    return pl.pallas_call(
        _embed_body,
        grid=grid,
        in_specs=[
            pl.BlockSpec(memory_space=pltpu.SMEM),
            pl.BlockSpec((1, ROWS, 1), lambda b, i: (b, i, 0)),
            pl.BlockSpec((1, ROWS, 3), lambda b, i: (b, i, 0)),
            pl.BlockSpec(tok.shape, lambda b, i: (0, 0)),
            pl.BlockSpec((ROWS, emb), lambda b, i: (i, 0)),
        ],
        out_specs=[
            pl.BlockSpec((1, ROWS, TBL), lambda b, i: (b, i, 0)),
            pl.BlockSpec((1, 3, ROWS), lambda b, i: (b, 0, i)),
        ],
        out_shape=[
            jax.ShapeDtypeStruct((B, L, TBL), jnp.float32),
            jax.ShapeDtypeStruct((B, 3, L), jnp.float32),
        ],
    )(lengths, res3, coords, tok, pos)


def _topk_body(len_ref, table_ref, ct_ref, idx_ref):
    b = pl.program_id(0)
    iblk = pl.program_id(1)
    L = ct_ref.shape[2]
    len_b = len_ref[b]

    @pl.when(iblk * ROWS >= len_b)
    def _skip():
        idx_ref[0] = jnp.full((K - 1, ROWS), b * L, jnp.int32)

    @pl.when(iblk * ROWS < len_b)
    def _full():
        t = table_ref[0]
        xr = t[:, 16:19]
        ct = ct_ref[0]
        ii = iblk * ROWS + lax.broadcasted_iota(jnp.int32, (ROWS, 1), 0)
        ji = lax.broadcasted_iota(jnp.int32, (1, L), 1)
        nx = (xr[:, 0:1] * xr[:, 0:1] + xr[:, 1:2] * xr[:, 1:2]
              + xr[:, 2:3] * xr[:, 2:3])
        nj = (ct[0:1, :] * ct[0:1, :] + ct[1:2, :] * ct[1:2, :]
              + ct[2:3, :] * ct[2:3, :])
        dist = (nx + nj) - 2.0 * jnp.dot(xr, ct,
                                         preferred_element_type=jnp.float32)
        valid = (ii < len_b) & (ji < len_b)
        rank = jnp.where(valid, dist, jnp.float32(1e5))
        rank = jnp.where(ii == ji, jnp.float32(jnp.inf), rank)
        adj = (jnp.abs(ii - ji) == 1) & (jnp.maximum(ii, ji) < len_b)
        rank = jnp.where(adj, jnp.float32(0.0), rank)
        jif = ji.astype(jnp.float32)
        cols = []
        for _ in range(K - 1):
            mval = jnp.min(rank, axis=1, keepdims=True)
            amin = jnp.min(jnp.where(rank == mval, jif, jnp.float32(L)),
                           axis=1, keepdims=True)
            cols.append(amin)
            rank = jnp.where(jif == amin, jnp.float32(jnp.inf), rank)
        A = jnp.concatenate(cols, axis=1) + jnp.float32(b * L)
        idx_ref[0] = jnp.swapaxes(A, 0, 1).astype(jnp.int32)


def _topk(lengths, table, ct):
    B, L, _ = table.shape
    grid = (B, L // ROWS)
    return pl.pallas_call(
        _topk_body,
        grid=grid,
        in_specs=[
            pl.BlockSpec(memory_space=pltpu.SMEM),
            pl.BlockSpec((1, ROWS, TBL), lambda b, i: (b, i, 0)),
            pl.BlockSpec((1, 3, L), lambda b, i: (b, 0, 0)),
        ],
        out_specs=pl.BlockSpec((1, K - 1, ROWS), lambda b, i: (b, 0, i)),
        out_shape=jax.ShapeDtypeStruct((B, K - 1, L), jnp.int32),
    )(lengths, table, ct)


def _sc_gather(table_flat, gidx):
    M = gidx.shape[0]
    N, D = table_flat.shape
    info = plsc.get_sparse_core_info()
    nw = info.num_cores * info.num_subcores
    rows_w = M // nw
    ch = 128
    nch = rows_w // ch
    stage = N // info.num_subcores
    mesh = plsc.VectorSubcoreMesh(core_axis_name="c", subcore_axis_name="s")

    @functools.partial(
        pl.kernel,
        out_type=jax.ShapeDtypeStruct((M, 128), jnp.float32),
        mesh=mesh,
        compiler_params=pltpu.CompilerParams(use_tc_tiling_on_sc=False),
        scratch_types=[
            pltpu.VMEM((rows_w,), jnp.int32),
            pltpu.VMEM((rows_w, D), jnp.float32),
            pltpu.VMEM_SHARED((N, D), jnp.float32),
            pltpu.SemaphoreType.DMA,
        ],
    )
    def gk(table_hbm, idx_hbm, out_hbm, idx_v, rows_v, shared, sem):
        sid = lax.axis_index("s")
        wid = sid * info.num_cores + lax.axis_index("c")
        base = wid * rows_w
        pltpu.sync_copy(table_hbm.at[pl.ds(sid * stage, stage)],
                        shared.at[pl.ds(sid * stage, stage)])
        pltpu.sync_copy(idx_hbm.at[pl.ds(base, rows_w)], idx_v)
        plsc.subcore_barrier()
        cps = [
            pltpu.async_copy(
                shared.at[idx_v.at[pl.ds(c * ch, ch)]],
                rows_v.at[pl.ds(c * ch, ch)],
                sem,
            )
            for c in range(nch)
        ]
        for cp in cps:
            cp.wait()
        pltpu.sync_copy(rows_v, out_hbm.at[pl.ds(base, rows_w), pl.ds(0, D)])

    return gk(table_flat, gidx)


def _mp_body(len_ref, table_ref, g_ref,
             ew1_ref, eb1_ref, ew2_ref, eb2_ref,
             nw1_ref, nb1_ref, nw2_ref, nb2_ref,
             cw1_ref, cb1_ref, cw2_ref, cb2_ref,
             lnw_ref, lnb_ref, *rest, last):
    if last:
        fw_ref, fb_ref, out_ref = rest
    else:
        fw_ref = fb_ref = None
        out_ref, ct_ref = rest
    b = pl.program_id(0)
    iblk = pl.program_id(1)
    len_b = len_ref[b]
    t = table_ref[0]
    feats = t[:, 0:16]
    coors = t[:, 16:19]
    R = feats.shape[0]

    def node_update(m_i, coors_new):
        mu = jnp.mean(feats, axis=1, keepdims=True)
        var = jnp.mean((feats - mu) ** 2, axis=1, keepdims=True)
        normed = ((feats - mu) / jnp.sqrt(var + 1e-5) * lnw_ref[...]
                  + lnb_ref[...])
        node_in = jnp.concatenate([normed, m_i], axis=1)
        h2 = _silu(jnp.dot(node_in, nw1_ref[...],
                           preferred_element_type=jnp.float32) + nb1_ref[...])
        node_out = (jnp.dot(h2, nw2_ref[...],
                            preferred_element_type=jnp.float32)
                    + nb2_ref[...] + feats)
        if last:
            out_ref[0] = (jnp.dot(node_out, fw_ref[...],
                                  preferred_element_type=jnp.float32)
                          + fb_ref[...])
        else:
            ii = iblk * R + lax.broadcasted_iota(jnp.int32, (R, 1), 0)
            maskc = (ii < len_b).astype(jnp.float32)
            pad = jnp.zeros((R, TBL - 20), jnp.float32)
            out_ref[0] = jnp.concatenate([node_out, coors_new, maskc, pad],
                                         axis=1)
            ct_ref[0] = jnp.swapaxes(coors_new, 0, 1)

    @pl.when(iblk * R >= len_b)
    def _skip():
        node_update(jnp.zeros_like(feats), coors)

    @pl.when(iblk * R < len_b)
    def _full():
        ii = iblk * R + lax.broadcasted_iota(jnp.int32, (R, 1), 0)
        mask_i = ii < len_b

        zero1 = jnp.zeros((R, 1), jnp.float32)
        e_parts = [jnp.concatenate([feats, feats, zero1], axis=1)]
        m2_parts = [mask_i]
        rel_parts = []
        for k in range(1, K):
            g = g_ref[k - 1]
            fj = g[:, 0:16]
            cj = g[:, 16:19]
            rel = coors - cj
            a = rel[:, 0:1]
            bb = rel[:, 1:2]
            c = rel[:, 2:3]
            rd = a * a + bb * bb + c * c
            e_parts.append(jnp.concatenate([feats, fj, rd], axis=1))
            m2_parts.append(mask_i & (g[:, 19:20] > 0.5))
            rel_parts.append(rel)
        E = jnp.concatenate(e_parts, axis=0)
        M2 = jnp.concatenate(m2_parts, axis=0)

        h = _silu(jnp.dot(E, ew1_ref[...], preferred_element_type=jnp.float32)
                  + eb1_ref[...])
        m = _silu(jnp.dot(h, ew2_ref[...], preferred_element_type=jnp.float32)
                  + eb2_ref[...])
        m = jnp.where(M2, m, jnp.float32(0.0))
        m_i = m[0:R]
        for k in range(1, K):
            m_i = m_i + m[k * R:(k + 1) * R]

        if last:
            coors_new = None
        else:
            REL = jnp.concatenate(rel_parts, axis=0)
            cw = jnp.dot(_silu(jnp.dot(m[R:], cw1_ref[...],
                                       preferred_element_type=jnp.float32)
                               + cb1_ref[...]),
                         cw2_ref[...],
                         preferred_element_type=jnp.float32) + cb2_ref[...]
            cw = jnp.where(M2[R:], cw, jnp.float32(0.0))
            contrib = cw * REL
            coors_new = coors
            for k in range(1, K):
                coors_new = coors_new + contrib[(k - 1) * R:k * R]
        node_update(m_i, coors_new)


def _mp(lengths, table, G, w, final):
    B, L, _ = table.shape
    grid = (B, L // ROWS)
    full = lambda a: pl.BlockSpec(a.shape, lambda b, i: (0,) * a.ndim)
    last = final is not None
    in_specs = [
        pl.BlockSpec(memory_space=pltpu.SMEM),
        pl.BlockSpec((1, ROWS, TBL), lambda b, i: (b, i, 0)),
        pl.BlockSpec((K - 1, ROWS, 128), lambda b, i: (b, i, 0)),
    ] + [full(a) for a in w]
    args = [lengths, table, G, *w]
    if last:
        in_specs += [full(a) for a in final]
        args += list(final)
        out_specs = pl.BlockSpec((1, ROWS, 3), lambda b, i: (b, i, 0))
        out_shape = jax.ShapeDtypeStruct((B, L, 3), jnp.float32)
    else:
        out_specs = [
            pl.BlockSpec((1, ROWS, TBL), lambda b, i: (b, i, 0)),
            pl.BlockSpec((1, 3, ROWS), lambda b, i: (b, 0, i)),
        ]
        out_shape = [
            jax.ShapeDtypeStruct((B, L, TBL), jnp.float32),
            jax.ShapeDtypeStruct((B, 3, L), jnp.float32),
        ]
    return pl.pallas_call(
        functools.partial(_mp_body, last=last),
        grid=grid,
        in_specs=in_specs,
        out_specs=out_specs,
        out_shape=out_shape,
    )(*args)


def kernel(coords, residues, lengths, params):
    B, L, _ = coords.shape
    lengths = lengths.astype(jnp.int32)
    table, ct = _embed(lengths, residues, coords.astype(jnp.float32),
                       params['token_emb'], params['pos_emb'])
    nlayers = len(params['layers'])
    out = None
    for li, lp in enumerate(params['layers']):
        idx_t = _topk(lengths, table, ct)
        G = _sc_gather(table.reshape(B * L, TBL),
                       idx_t.reshape(-1)).reshape(B * (K - 1), L, 128)
        w = (lp['e_w1'], lp['e_b1'].reshape(1, -1),
             lp['e_w2'], lp['e_b2'].reshape(1, -1),
             lp['n_w1'], lp['n_b1'].reshape(1, -1),
             lp['n_w2'], lp['n_b2'].reshape(1, -1),
             lp['c_w1'], lp['c_b1'].reshape(1, -1),
             lp['c_w2'], lp['c_b2'].reshape(1, -1),
             lp['ln_w'].reshape(1, -1), lp['ln_b'].reshape(1, -1))
        if li == nlayers - 1:
            out = _mp(lengths, table, G, w,
                      (params['final_w'], params['final_b'].reshape(1, -1)))
        else:
            table, ct = _mp(lengths, table, G, w, None)
    return out


# speedup vs baseline: 1.2955x; 1.1640x over previous
import functools

import jax
import jax.numpy as jnp
from jax import lax
from jax.experimental import pallas as pl
from jax.experimental.pallas import tpu as pltpu
from jax.experimental.pallas import tpu_sc as plsc

K = 8
ROWS = 512
TBL = 32


def _silu(x):
    return x * jax.nn.sigmoid(x)


def _embed_body(len_ref, res_ref, coords_ref, tok_ref, pos_ref,
                out_ref, ct_ref):
    b = pl.program_id(0)
    iblk = pl.program_id(1)
    len_b = len_ref[b]
    res = res_ref[0]
    ntok = tok_ref.shape[0]
    f = jnp.where(res == ntok - 2, tok_ref[ntok - 2:ntok - 1, :],
                  tok_ref[ntok - 1:ntok, :])
    for t in range(ntok - 3, -1, -1):
        f = jnp.where(res == t, tok_ref[t:t + 1, :], f)
    feats = f + pos_ref[...]
    R = feats.shape[0]
    ii = iblk * R + lax.broadcasted_iota(jnp.int32, (R, 1), 0)
    maskc = (ii < len_b).astype(jnp.float32)
    pad = jnp.zeros((R, TBL - 20), jnp.float32)
    coors = coords_ref[0]
    out_ref[0] = jnp.concatenate([feats, coors, maskc, pad], axis=1)
    ct_ref[0] = jnp.swapaxes(coors, 0, 1)


def _embed(lengths, residues, coords, tok, pos):
    B, L = residues.shape
    emb = tok.shape[1]
    res3 = residues.reshape(B, L, 1).astype(jnp.int32)
    grid = (B, L // ROWS)
    return pl.pallas_call(
        _embed_body,
        grid=grid,
        in_specs=[
            pl.BlockSpec(memory_space=pltpu.SMEM),
            pl.BlockSpec((1, ROWS, 1), lambda b, i: (b, i, 0)),
            pl.BlockSpec((1, ROWS, 3), lambda b, i: (b, i, 0)),
            pl.BlockSpec(tok.shape, lambda b, i: (0, 0)),
            pl.BlockSpec((ROWS, emb), lambda b, i: (i, 0)),
        ],
        out_specs=[
            pl.BlockSpec((1, ROWS, TBL), lambda b, i: (b, i, 0)),
            pl.BlockSpec((1, 3, ROWS), lambda b, i: (b, 0, i)),
        ],
        out_shape=[
            jax.ShapeDtypeStruct((B, L, TBL), jnp.float32),
            jax.ShapeDtypeStruct((B, 3, L), jnp.float32),
        ],
    )(lengths, res3, coords, tok, pos)


def _topk_body(len_ref, table_ref, ct_ref, idx_ref):
    b = pl.program_id(0)
    iblk = pl.program_id(1)
    L = ct_ref.shape[2]
    len_b = len_ref[b]

    @pl.when(iblk * ROWS >= len_b)
    def _skip():
        idx_ref[0] = jnp.full((K - 1, ROWS), b * L, jnp.int32)

    def extract(W):
        t = table_ref[0]
        xr = t[:, 16:19]
        ct = ct_ref[0][:, 0:W]
        ii = iblk * ROWS + lax.broadcasted_iota(jnp.int32, (ROWS, 1), 0)
        ji = lax.broadcasted_iota(jnp.int32, (1, W), 1)
        nx = (xr[:, 0:1] * xr[:, 0:1] + xr[:, 1:2] * xr[:, 1:2]
              + xr[:, 2:3] * xr[:, 2:3])
        nj = (ct[0:1, :] * ct[0:1, :] + ct[1:2, :] * ct[1:2, :]
              + ct[2:3, :] * ct[2:3, :])
        dist = (nx + nj) - 2.0 * jnp.dot(xr, ct,
                                         preferred_element_type=jnp.float32)
        valid = (ii < len_b) & (ji < len_b)
        rank = jnp.where(valid, dist, jnp.float32(1e5))
        rank = jnp.where(ii == ji, jnp.float32(jnp.inf), rank)
        adj = (jnp.abs(ii - ji) == 1) & (jnp.maximum(ii, ji) < len_b)
        rank = jnp.where(adj, jnp.float32(0.0), rank)
        jif = ji.astype(jnp.float32)
        cols = []
        for _ in range(K - 1):
            mval = jnp.min(rank, axis=1, keepdims=True)
            amin = jnp.min(jnp.where(rank == mval, jif, jnp.float32(W)),
                           axis=1, keepdims=True)
            cols.append(amin)
            rank = jnp.where(jif == amin, jnp.float32(jnp.inf), rank)
        A = jnp.concatenate(cols, axis=1) + jnp.float32(b * L)
        idx_ref[0] = jnp.swapaxes(A, 0, 1).astype(jnp.int32)

    wtiles = (len_b + 511) // 512
    for tix in range(1, L // 512 + 1):
        @pl.when((iblk * ROWS < len_b) & (wtiles == tix))
        def _branch(tix=tix):
            extract(512 * tix)


def _topk(lengths, table, ct):
    B, L, _ = table.shape
    grid = (B, L // ROWS)
    return pl.pallas_call(
        _topk_body,
        grid=grid,
        in_specs=[
            pl.BlockSpec(memory_space=pltpu.SMEM),
            pl.BlockSpec((1, ROWS, TBL), lambda b, i: (b, i, 0)),
            pl.BlockSpec((1, 3, L), lambda b, i: (b, 0, 0)),
        ],
        out_specs=pl.BlockSpec((1, K - 1, ROWS), lambda b, i: (b, 0, i)),
        out_shape=jax.ShapeDtypeStruct((B, K - 1, L), jnp.int32),
    )(lengths, table, ct)


def _sc_gather(table_flat, gidx):
    M = gidx.shape[0]
    N, D = table_flat.shape
    info = plsc.get_sparse_core_info()
    nw = info.num_cores * info.num_subcores
    rows_w = M // nw
    ch = 128
    nch = rows_w // ch
    stage = N // info.num_subcores
    mesh = plsc.VectorSubcoreMesh(core_axis_name="c", subcore_axis_name="s")

    @functools.partial(
        pl.kernel,
        out_type=jax.ShapeDtypeStruct((M, 128), jnp.float32),
        mesh=mesh,
        compiler_params=pltpu.CompilerParams(use_tc_tiling_on_sc=False),
        scratch_types=[
            pltpu.VMEM((rows_w,), jnp.int32),
            pltpu.VMEM((rows_w, D), jnp.float32),
            pltpu.VMEM_SHARED((N, D), jnp.float32),
            pltpu.SemaphoreType.DMA,
        ],
    )
    def gk(table_hbm, idx_hbm, out_hbm, idx_v, rows_v, shared, sem):
        sid = lax.axis_index("s")
        wid = sid * info.num_cores + lax.axis_index("c")
        base = wid * rows_w
        pltpu.sync_copy(table_hbm.at[pl.ds(sid * stage, stage)],
                        shared.at[pl.ds(sid * stage, stage)])
        pltpu.sync_copy(idx_hbm.at[pl.ds(base, rows_w)], idx_v)
        plsc.subcore_barrier()
        cps = [
            pltpu.async_copy(
                shared.at[idx_v.at[pl.ds(c * ch, ch)]],
                rows_v.at[pl.ds(c * ch, ch)],
                sem,
            )
            for c in range(nch)
        ]
        for cp in cps:
            cp.wait()
        pltpu.sync_copy(rows_v, out_hbm.at[pl.ds(base, rows_w), pl.ds(0, D)])

    return gk(table_flat, gidx)


def _mp_body(len_ref, table_ref, g_ref,
             ew1_ref, eb1_ref, ew2_ref, eb2_ref,
             nw1_ref, nb1_ref, nw2_ref, nb2_ref,
             cw1_ref, cb1_ref, cw2_ref, cb2_ref,
             lnw_ref, lnb_ref, *rest, last):
    if last:
        fw_ref, fb_ref, out_ref = rest
    else:
        fw_ref = fb_ref = None
        out_ref, ct_ref = rest
    b = pl.program_id(0)
    iblk = pl.program_id(1)
    len_b = len_ref[b]
    t = table_ref[0]
    feats = t[:, 0:16]
    coors = t[:, 16:19]
    R = feats.shape[0]

    def node_update(m_i, coors_new):
        mu = jnp.mean(feats, axis=1, keepdims=True)
        var = jnp.mean((feats - mu) ** 2, axis=1, keepdims=True)
        normed = ((feats - mu) / jnp.sqrt(var + 1e-5) * lnw_ref[...]
                  + lnb_ref[...])
        node_in = jnp.concatenate([normed, m_i], axis=1)
        h2 = _silu(jnp.dot(node_in, nw1_ref[...],
                           preferred_element_type=jnp.float32) + nb1_ref[...])
        node_out = (jnp.dot(h2, nw2_ref[...],
                            preferred_element_type=jnp.float32)
                    + nb2_ref[...] + feats)
        if last:
            out_ref[0] = (jnp.dot(node_out, fw_ref[...],
                                  preferred_element_type=jnp.float32)
                          + fb_ref[...])
        else:
            ii = iblk * R + lax.broadcasted_iota(jnp.int32, (R, 1), 0)
            maskc = (ii < len_b).astype(jnp.float32)
            pad = jnp.zeros((R, TBL - 20), jnp.float32)
            out_ref[0] = jnp.concatenate([node_out, coors_new, maskc, pad],
                                         axis=1)
            ct_ref[0] = jnp.swapaxes(coors_new, 0, 1)

    @pl.when(iblk * R >= len_b)
    def _skip():
        node_update(jnp.zeros_like(feats), coors)

    @pl.when(iblk * R < len_b)
    def _full():
        ii = iblk * R + lax.broadcasted_iota(jnp.int32, (R, 1), 0)
        mask_i = ii < len_b

        zero1 = jnp.zeros((R, 1), jnp.float32)
        e_parts = [jnp.concatenate([feats, feats, zero1], axis=1)]
        m2_parts = [mask_i]
        rel_parts = []
        for k in range(1, K):
            g = g_ref[k - 1]
            fj = g[:, 0:16]
            cj = g[:, 16:19]
            rel = coors - cj
            a = rel[:, 0:1]
            bb = rel[:, 1:2]
            c = rel[:, 2:3]
            rd = a * a + bb * bb + c * c
            e_parts.append(jnp.concatenate([feats, fj, rd], axis=1))
            m2_parts.append(mask_i & (g[:, 19:20] > 0.5))
            rel_parts.append(rel)
        E = jnp.concatenate(e_parts, axis=0)
        M2 = jnp.concatenate(m2_parts, axis=0)

        h = _silu(jnp.dot(E, ew1_ref[...], preferred_element_type=jnp.float32)
                  + eb1_ref[...])
        m = _silu(jnp.dot(h, ew2_ref[...], preferred_element_type=jnp.float32)
                  + eb2_ref[...])
        m = jnp.where(M2, m, jnp.float32(0.0))
        m_i = m[0:R]
        for k in range(1, K):
            m_i = m_i + m[k * R:(k + 1) * R]

        if last:
            coors_new = None
        else:
            REL = jnp.concatenate(rel_parts, axis=0)
            cw = jnp.dot(_silu(jnp.dot(m[R:], cw1_ref[...],
                                       preferred_element_type=jnp.float32)
                               + cb1_ref[...]),
                         cw2_ref[...],
                         preferred_element_type=jnp.float32) + cb2_ref[...]
            cw = jnp.where(M2[R:], cw, jnp.float32(0.0))
            contrib = cw * REL
            coors_new = coors
            for k in range(1, K):
                coors_new = coors_new + contrib[(k - 1) * R:k * R]
        node_update(m_i, coors_new)


def _mp(lengths, table, G, w, final):
    B, L, _ = table.shape
    grid = (B, L // ROWS)
    full = lambda a: pl.BlockSpec(a.shape, lambda b, i: (0,) * a.ndim)
    last = final is not None
    in_specs = [
        pl.BlockSpec(memory_space=pltpu.SMEM),
        pl.BlockSpec((1, ROWS, TBL), lambda b, i: (b, i, 0)),
        pl.BlockSpec((K - 1, ROWS, 128), lambda b, i: (b, i, 0)),
    ] + [full(a) for a in w]
    args = [lengths, table, G, *w]
    if last:
        in_specs += [full(a) for a in final]
        args += list(final)
        out_specs = pl.BlockSpec((1, ROWS, 3), lambda b, i: (b, i, 0))
        out_shape = jax.ShapeDtypeStruct((B, L, 3), jnp.float32)
    else:
        out_specs = [
            pl.BlockSpec((1, ROWS, TBL), lambda b, i: (b, i, 0)),
            pl.BlockSpec((1, 3, ROWS), lambda b, i: (b, 0, i)),
        ]
        out_shape = [
            jax.ShapeDtypeStruct((B, L, TBL), jnp.float32),
            jax.ShapeDtypeStruct((B, 3, L), jnp.float32),
        ]
    return pl.pallas_call(
        functools.partial(_mp_body, last=last),
        grid=grid,
        in_specs=in_specs,
        out_specs=out_specs,
        out_shape=out_shape,
    )(*args)


def kernel(coords, residues, lengths, params):
    B, L, _ = coords.shape
    lengths = lengths.astype(jnp.int32)
    table, ct = _embed(lengths, residues, coords.astype(jnp.float32),
                       params['token_emb'], params['pos_emb'])
    nlayers = len(params['layers'])
    out = None
    for li, lp in enumerate(params['layers']):
        idx_t = _topk(lengths, table, ct)
        G = _sc_gather(table.reshape(B * L, TBL),
                       idx_t.reshape(-1)).reshape(B * (K - 1), L, 128)
        w = (lp['e_w1'], lp['e_b1'].reshape(1, -1),
             lp['e_w2'], lp['e_b2'].reshape(1, -1),
             lp['n_w1'], lp['n_b1'].reshape(1, -1),
             lp['n_w2'], lp['n_b2'].reshape(1, -1),
             lp['c_w1'], lp['c_b1'].reshape(1, -1),
             lp['c_w2'], lp['c_b2'].reshape(1, -1),
             lp['ln_w'].reshape(1, -1), lp['ln_b'].reshape(1, -1))
        if li == nlayers - 1:
            out = _mp(lengths, table, G, w,
                      (params['final_w'], params['final_b'].reshape(1, -1)))
        else:
            table, ct = _mp(lengths, table, G, w, None)
    return out


# speedup vs baseline: 1.4086x; 1.0873x over previous
import functools

import jax
import jax.numpy as jnp
from jax import lax
from jax.experimental import pallas as pl
from jax.experimental.pallas import tpu as pltpu
from jax.experimental.pallas import tpu_sc as plsc

K = 8
ROWS = 512
TBL = 32


def _silu(x):
    return x * jax.nn.sigmoid(x)


def _embed_body(len_ref, res_ref, coords_ref, tok_ref, pos_ref,
                out_ref, ct_ref):
    b = pl.program_id(0)
    iblk = pl.program_id(1)
    len_b = len_ref[b]
    res = res_ref[0]
    ntok = tok_ref.shape[0]
    f = jnp.where(res == ntok - 2, tok_ref[ntok - 2:ntok - 1, :],
                  tok_ref[ntok - 1:ntok, :])
    for t in range(ntok - 3, -1, -1):
        f = jnp.where(res == t, tok_ref[t:t + 1, :], f)
    feats = f + pos_ref[...]
    R = feats.shape[0]
    ii = iblk * R + lax.broadcasted_iota(jnp.int32, (R, 1), 0)
    maskc = (ii < len_b).astype(jnp.float32)
    pad = jnp.zeros((R, TBL - 20), jnp.float32)
    coors = coords_ref[0]
    out_ref[0] = jnp.concatenate([feats, coors, maskc, pad], axis=1)
    ct_ref[0] = jnp.swapaxes(coors, 0, 1)


def _embed(lengths, residues, coords, tok, pos):
    B, L = residues.shape
    emb = tok.shape[1]
    res3 = residues.reshape(B, L, 1).astype(jnp.int32)
    grid = (B, L // ROWS)
    return pl.pallas_call(
        _embed_body,
        grid=grid,
        in_specs=[
            pl.BlockSpec(memory_space=pltpu.SMEM),
            pl.BlockSpec((1, ROWS, 1), lambda b, i: (b, i, 0)),
            pl.BlockSpec((1, ROWS, 3), lambda b, i: (b, i, 0)),
            pl.BlockSpec(tok.shape, lambda b, i: (0, 0)),
            pl.BlockSpec((ROWS, emb), lambda b, i: (i, 0)),
        ],
        out_specs=[
            pl.BlockSpec((1, ROWS, TBL), lambda b, i: (b, i, 0)),
            pl.BlockSpec((1, 3, ROWS), lambda b, i: (b, 0, i)),
        ],
        out_shape=[
            jax.ShapeDtypeStruct((B, L, TBL), jnp.float32),
            jax.ShapeDtypeStruct((B, 3, L), jnp.float32),
        ],
    )(lengths, res3, coords, tok, pos)


def _topk_body(len_ref, table_ref, ct_ref, idx_ref):
    b = pl.program_id(0)
    iblk = pl.program_id(1)
    L = ct_ref.shape[2]
    len_b = len_ref[b]

    @pl.when(iblk * ROWS >= len_b)
    def _skip():
        idx_ref[0] = jnp.full((K - 1, ROWS), b * L, jnp.int32)

    def extract(W):
        t = table_ref[0]
        xr = t[:, 16:19]
        ct = ct_ref[0][:, 0:W]
        ii = iblk * ROWS + lax.broadcasted_iota(jnp.int32, (ROWS, 1), 0)
        ji = lax.broadcasted_iota(jnp.int32, (1, W), 1)
        nx = (xr[:, 0:1] * xr[:, 0:1] + xr[:, 1:2] * xr[:, 1:2]
              + xr[:, 2:3] * xr[:, 2:3])
        nj = (ct[0:1, :] * ct[0:1, :] + ct[1:2, :] * ct[1:2, :]
              + ct[2:3, :] * ct[2:3, :])
        dist = (nx + nj) - 2.0 * jnp.dot(xr, ct,
                                         preferred_element_type=jnp.float32)
        valid = (ii < len_b) & (ji < len_b)
        rank = jnp.where(valid, dist, jnp.float32(1e5))
        rank = jnp.where(ii == ji, jnp.float32(jnp.inf), rank)
        adj = (jnp.abs(ii - ji) == 1) & (jnp.maximum(ii, ji) < len_b)
        rank = jnp.where(adj, jnp.float32(0.0), rank)
        jif = ji.astype(jnp.float32)
        cols = []
        for _ in range(K - 1):
            mval = jnp.min(rank, axis=1, keepdims=True)
            amin = jnp.min(jnp.where(rank == mval, jif, jnp.float32(W)),
                           axis=1, keepdims=True)
            cols.append(amin)
            rank = jnp.where(jif == amin, jnp.float32(jnp.inf), rank)
        A = jnp.concatenate(cols, axis=1) + jnp.float32(b * L)
        idx_ref[0] = jnp.swapaxes(A, 0, 1).astype(jnp.int32)

    wtiles = (len_b + 511) // 512
    for tix in range(1, L // 512 + 1):
        @pl.when((iblk * ROWS < len_b) & (wtiles == tix))
        def _branch(tix=tix):
            extract(512 * tix)


def _topk(lengths, table, ct):
    B, L, _ = table.shape
    grid = (B, L // ROWS)
    return pl.pallas_call(
        _topk_body,
        grid=grid,
        in_specs=[
            pl.BlockSpec(memory_space=pltpu.SMEM),
            pl.BlockSpec((1, ROWS, TBL), lambda b, i: (b, i, 0)),
            pl.BlockSpec((1, 3, L), lambda b, i: (b, 0, 0)),
        ],
        out_specs=pl.BlockSpec((1, K - 1, ROWS), lambda b, i: (b, 0, i)),
        out_shape=jax.ShapeDtypeStruct((B, K - 1, L), jnp.int32),
    )(lengths, table, ct)


def _sc_gather(table_flat, gidx):
    M = gidx.shape[0]
    N, D = table_flat.shape
    info = plsc.get_sparse_core_info()
    nw = info.num_cores * info.num_subcores
    rows_w = M // nw
    ch = 128
    nch = rows_w // ch
    stage = N // info.num_subcores
    mesh = plsc.VectorSubcoreMesh(core_axis_name="c", subcore_axis_name="s")

    @functools.partial(
        pl.kernel,
        out_type=jax.ShapeDtypeStruct((M, 128), jnp.float32),
        mesh=mesh,
        compiler_params=pltpu.CompilerParams(use_tc_tiling_on_sc=False),
        scratch_types=[
            pltpu.VMEM((rows_w,), jnp.int32),
            pltpu.VMEM((rows_w, D), jnp.float32),
            pltpu.VMEM_SHARED((N, D), jnp.float32),
            pltpu.SemaphoreType.DMA,
        ],
    )
    def gk(table_hbm, idx_hbm, out_hbm, idx_v, rows_v, shared, sem):
        sid = lax.axis_index("s")
        wid = sid * info.num_cores + lax.axis_index("c")
        base = wid * rows_w
        pltpu.sync_copy(table_hbm.at[pl.ds(sid * stage, stage)],
                        shared.at[pl.ds(sid * stage, stage)])
        pltpu.sync_copy(idx_hbm.at[pl.ds(base, rows_w)], idx_v)
        plsc.subcore_barrier()
        cps = [
            pltpu.async_copy(
                shared.at[idx_v.at[pl.ds(c * ch, ch)]],
                rows_v.at[pl.ds(c * ch, ch)],
                sem,
            )
            for c in range(nch)
        ]
        for cp in cps:
            cp.wait()
        pltpu.sync_copy(rows_v, out_hbm.at[pl.ds(base, rows_w), pl.ds(0, D)])

    return gk(table_flat, gidx)


def _mp_body(len_ref, table_ref, g_ref,
             ew1_ref, eb1_ref, ew2_ref, eb2_ref,
             nw1_ref, nb1_ref, nw2_ref, nb2_ref,
             cw1_ref, cb1_ref, cw2_ref, cb2_ref,
             lnw_ref, lnb_ref, *rest, last):
    if last:
        fw_ref, fb_ref, out_ref = rest
    else:
        fw_ref = fb_ref = None
        out_ref, ct_ref = rest
    b = pl.program_id(0)
    iblk = pl.program_id(1)
    len_b = len_ref[b]
    t = table_ref[0]
    feats = t[:, 0:16]
    coors = t[:, 16:19]
    R = feats.shape[0]

    def node_update(m_i, coors_new):
        mu = jnp.mean(feats, axis=1, keepdims=True)
        var = jnp.mean((feats - mu) ** 2, axis=1, keepdims=True)
        normed = ((feats - mu) / jnp.sqrt(var + 1e-5) * lnw_ref[...]
                  + lnb_ref[...])
        d = normed.shape[1]
        h2 = _silu(jnp.dot(normed, nw1_ref[0:d, :],
                           preferred_element_type=jnp.float32)
                   + jnp.dot(m_i, nw1_ref[d:2 * d, :],
                             preferred_element_type=jnp.float32)
                   + nb1_ref[...])
        node_out = (jnp.dot(h2, nw2_ref[...],
                            preferred_element_type=jnp.float32)
                    + nb2_ref[...] + feats)
        if last:
            out_ref[0] = (jnp.dot(node_out, fw_ref[...],
                                  preferred_element_type=jnp.float32)
                          + fb_ref[...])
        else:
            ii = iblk * R + lax.broadcasted_iota(jnp.int32, (R, 1), 0)
            maskc = (ii < len_b).astype(jnp.float32)
            pad = jnp.zeros((R, TBL - 20), jnp.float32)
            out_ref[0] = jnp.concatenate([node_out, coors_new, maskc, pad],
                                         axis=1)
            ct_ref[0] = jnp.swapaxes(coors_new, 0, 1)

    @pl.when(iblk * R >= len_b)
    def _skip():
        node_update(jnp.zeros_like(feats), coors)

    @pl.when(iblk * R < len_b)
    def _full():
        ii = iblk * R + lax.broadcasted_iota(jnp.int32, (R, 1), 0)
        mask_i = ii < len_b
        d = feats.shape[1]

        w1a = ew1_ref[0:d, :]
        w1b = ew1_ref[d:2 * d, :]
        w1c = ew1_ref[2 * d:2 * d + 1, :]
        hi = (jnp.dot(feats, w1a, preferred_element_type=jnp.float32)
              + eb1_ref[...])
        hs = jnp.dot(feats, w1b, preferred_element_type=jnp.float32)

        def edge(h1, m2):
            m_k = _silu(jnp.dot(_silu(h1), ew2_ref[...],
                                preferred_element_type=jnp.float32)
                        + eb2_ref[...])
            return jnp.where(m2, m_k, jnp.float32(0.0))

        m_i = edge(hi + hs, mask_i)
        coors_new = coors
        for k in range(1, K):
            g = g_ref[k - 1]
            fj = g[:, 0:16]
            cj = g[:, 16:19]
            rel = coors - cj
            a = rel[:, 0:1]
            bb = rel[:, 1:2]
            c = rel[:, 2:3]
            rd = a * a + bb * bb + c * c
            m2 = mask_i & (g[:, 19:20] > 0.5)
            m_k = edge(hi + jnp.dot(fj, w1b,
                                    preferred_element_type=jnp.float32)
                       + rd * w1c, m2)
            m_i = m_i + m_k
            if not last:
                cw = (jnp.dot(_silu(jnp.dot(m_k, cw1_ref[...],
                                            preferred_element_type=jnp.float32)
                                    + cb1_ref[...]),
                              cw2_ref[...],
                              preferred_element_type=jnp.float32)
                      + cb2_ref[...])
                cw = jnp.where(m2, cw, jnp.float32(0.0))
                coors_new = coors_new + cw * rel
        if last:
            coors_new = None
        node_update(m_i, coors_new)


def _mp(lengths, table, G, w, final):
    B, L, _ = table.shape
    grid = (B, L // ROWS)
    full = lambda a: pl.BlockSpec(a.shape, lambda b, i: (0,) * a.ndim)
    last = final is not None
    in_specs = [
        pl.BlockSpec(memory_space=pltpu.SMEM),
        pl.BlockSpec((1, ROWS, TBL), lambda b, i: (b, i, 0)),
        pl.BlockSpec((K - 1, ROWS, 128), lambda b, i: (b, i, 0)),
    ] + [full(a) for a in w]
    args = [lengths, table, G, *w]
    if last:
        in_specs += [full(a) for a in final]
        args += list(final)
        out_specs = pl.BlockSpec((1, ROWS, 3), lambda b, i: (b, i, 0))
        out_shape = jax.ShapeDtypeStruct((B, L, 3), jnp.float32)
    else:
        out_specs = [
            pl.BlockSpec((1, ROWS, TBL), lambda b, i: (b, i, 0)),
            pl.BlockSpec((1, 3, ROWS), lambda b, i: (b, 0, i)),
        ]
        out_shape = [
            jax.ShapeDtypeStruct((B, L, TBL), jnp.float32),
            jax.ShapeDtypeStruct((B, 3, L), jnp.float32),
        ]
    return pl.pallas_call(
        functools.partial(_mp_body, last=last),
        grid=grid,
        in_specs=in_specs,
        out_specs=out_specs,
        out_shape=out_shape,
    )(*args)


def kernel(coords, residues, lengths, params):
    B, L, _ = coords.shape
    lengths = lengths.astype(jnp.int32)
    table, ct = _embed(lengths, residues, coords.astype(jnp.float32),
                       params['token_emb'], params['pos_emb'])
    nlayers = len(params['layers'])
    out = None
    for li, lp in enumerate(params['layers']):
        idx_t = _topk(lengths, table, ct)
        G = _sc_gather(table.reshape(B * L, TBL),
                       idx_t.reshape(-1)).reshape(B * (K - 1), L, 128)
        w = (lp['e_w1'], lp['e_b1'].reshape(1, -1),
             lp['e_w2'], lp['e_b2'].reshape(1, -1),
             lp['n_w1'], lp['n_b1'].reshape(1, -1),
             lp['n_w2'], lp['n_b2'].reshape(1, -1),
             lp['c_w1'], lp['c_b1'].reshape(1, -1),
             lp['c_w2'], lp['c_b2'].reshape(1, -1),
             lp['ln_w'].reshape(1, -1), lp['ln_b'].reshape(1, -1))
        if li == nlayers - 1:
            out = _mp(lengths, table, G, w,
                      (params['final_w'], params['final_b'].reshape(1, -1)))
        else:
            table, ct = _mp(lengths, table, G, w, None)
    return out
